# Initial kernel scaffold; baseline (speedup 1.0000x reference)
import jax
import jax.numpy as jnp
from jax.experimental import pallas as pl


def kernel(x, edge_index, W1, b1, g1, be1, W2, b2, g2, be2):
    raise NotImplementedError("write your pallas kernel here")


# speedup vs baseline: 9.6826x; 9.6826x over previous
import functools

import jax
import jax.numpy as jnp
from jax import lax
from jax.experimental import pallas as pl
from jax.experimental.pallas import tpu as pltpu
from jax.experimental.pallas import tpu_sc as plsc

N = 50000
E = 800000
NP = 51200
BR = 512
NBR = NP // BR
RPT = NP // 16
CH = 128
G = 2
NCHP = 6400
EP = NCHP * CH
DUMP = NP - N
DEG_CPS = NCHP // 32
F32 = jnp.float32
EPS = 1e-5


def _sc_mesh():
    return plsc.VectorSubcoreMesh(core_axis_name="c", subcore_axis_name="s",
                                  num_cores=2, num_subcores=16)


def _deg_hist(dstm):
    @functools.partial(
        pl.kernel,
        out_type=jax.ShapeDtypeStruct((2, 1, NP), F32),
        mesh=_sc_mesh(),
        compiler_params=pltpu.CompilerParams(use_tc_tiling_on_sc=False),
        scratch_types=[
            pltpu.VMEM((2, G, CH), jnp.int32),
            pltpu.VMEM((CH,), F32),
            pltpu.VMEM((RPT,), F32),
            pltpu.VMEM_SHARED((NP,), F32),
            pltpu.SemaphoreType.DMA((2, G)),
        ],
    )
    def k(dstm_hbm, hist_hbm, dbuf, ones, zbuf, acc, ssem):
        c = lax.axis_index("c")
        s = lax.axis_index("s")

        @pl.loop(0, CH, step=16)
        def _(i):
            ones[pl.ds(i, 16)] = jnp.ones((16,), F32)

        @pl.loop(0, RPT, step=16)
        def _(i):
            zbuf[pl.ds(i, 16)] = jnp.zeros((16,), F32)

        pltpu.sync_copy(zbuf, acc.at[pl.ds(s * RPT, RPT)])
        plsc.subcore_barrier()

        off0 = (c * 16 + s) * DEG_CPS

        def scat_wait(kb, b):
            pltpu.make_async_copy(ones, acc.at[dbuf.at[kb, b]],
                                  ssem.at[kb, b]).wait()

        def group(kk, kb, warm):
            if warm is None:
                for b in range(G):
                    scat_wait(kb, b)
            elif warm is not False:
                @pl.when(warm)
                def _():
                    for b in range(G):
                        scat_wait(kb, b)
            pltpu.sync_copy(dstm_hbm.at[pl.ds(off0 + kk * G, G)], dbuf.at[kb])
            for b in range(G):
                pltpu.async_copy(ones, acc.at[dbuf.at[kb, b]],
                                 ssem.at[kb, b], add=True)

        @pl.loop(0, DEG_CPS // G, step=2)
        def _(kk):
            group(kk, 0, kk >= 2)
            group(kk + 1, 1, kk + 1 >= 2)

        for kb in range(2):
            for b in range(G):
                scat_wait(kb, b)
        plsc.subcore_barrier()
        pltpu.sync_copy(acc.at[pl.ds(s * RPT, RPT)],
                        hist_hbm.at[c, 0, pl.ds(s * RPT, RPT)])

    return k(dstm)


def _segment_sum(y, srcm, dstm, parts):
    passes = parts // 2
    cps = NCHP // 16
    ngrp = cps // G

    @functools.partial(
        pl.kernel,
        out_type=jax.ShapeDtypeStruct((parts, NP, 32), F32),
        mesh=_sc_mesh(),
        compiler_params=pltpu.CompilerParams(use_tc_tiling_on_sc=False),
        scratch_types=[
            pltpu.VMEM((2, G, CH), jnp.int32),
            pltpu.VMEM((2, G, CH), jnp.int32),
            pltpu.VMEM((2, G, CH, 32), F32),
            pltpu.VMEM((RPT // 32, 32), F32),
            pltpu.VMEM_SHARED((NP, 32), F32),
            pltpu.SemaphoreType.DMA((2, G)),
            pltpu.SemaphoreType.DMA((2, G)),
        ],
    )
    def k(y_hbm, srcm_hbm, dstm_hbm, out_hbm,
          sbuf, dbuf, rbuf, zbuf, acc, gsem, ssem):
        c = lax.axis_index("c")
        s = lax.axis_index("s")

        @pl.loop(0, RPT // 32)
        def _(r):
            zbuf[r, pl.ds(0, 16)] = jnp.zeros((16,), F32)
            zbuf[r, pl.ds(16, 16)] = jnp.zeros((16,), F32)

        off0 = s * cps

        for p_i in range(passes):
            part = c * passes + p_i

            for q in range(32):
                pltpu.sync_copy(
                    zbuf, acc.at[pl.ds(s * RPT + q * (RPT // 32), RPT // 32)])
            plsc.subcore_barrier()

            def scat_wait(kb, b):
                pltpu.make_async_copy(rbuf.at[kb, b], acc.at[dbuf.at[kb, b]],
                                      ssem.at[kb, b]).wait()

            def group(kk, kb, warm):
                if warm is None:
                    for b in range(G):
                        scat_wait(kb, b)
                elif warm is not False:
                    @pl.when(warm)
                    def _():
                        for b in range(G):
                            scat_wait(kb, b)
                ch0 = off0 + kk * G
                pltpu.sync_copy(srcm_hbm.at[pl.ds(ch0, G)], sbuf.at[kb])
                pltpu.sync_copy(dstm_hbm.at[pl.ds(ch0, G)], dbuf.at[kb])
                for b in range(G):
                    pltpu.async_copy(y_hbm.at[part].at[sbuf.at[kb, b]],
                                     rbuf.at[kb, b], gsem.at[kb, b])
                for b in range(G):
                    pltpu.make_async_copy(y_hbm.at[part].at[sbuf.at[kb, b]],
                                          rbuf.at[kb, b], gsem.at[kb, b]).wait()
                    pltpu.async_copy(rbuf.at[kb, b], acc.at[dbuf.at[kb, b]],
                                     ssem.at[kb, b], add=True)

            @pl.loop(0, ngrp, step=2)
            def _(kk):
                group(kk, 0, kk >= 2)
                group(kk + 1, 1, kk + 1 >= 2)

            for kb in range(2):
                for b in range(G):
                    scat_wait(kb, b)
            plsc.subcore_barrier()
            pltpu.sync_copy(acc.at[pl.ds(s * RPT, RPT)],
                            out_hbm.at[part, pl.ds(s * RPT, RPT)])

    return k(y, srcm, dstm)


def _xw1(x_p, w1p):
    def body(x_ref, w_ref, o_ref):
        o_ref[0] = jnp.dot(x_ref[...], w_ref[0],
                           preferred_element_type=F32)

    return pl.pallas_call(
        body,
        grid=(2, NBR),
        in_specs=[
            pl.BlockSpec((BR, 32), lambda p, i: (i, 0)),
            pl.BlockSpec((1, 32, 32), lambda p, i: (p, 0, 0)),
        ],
        out_specs=pl.BlockSpec((1, BR, 32), lambda p, i: (p, i, 0)),
        out_shape=jax.ShapeDtypeStruct((2, NP, 32), F32),
    )(x_p, w1p)


def _scale_y1(xw1, hist_t):
    def body(xw_ref, ht_ref, o_ref):
        dinv = lax.rsqrt(1.0 + ht_ref[:, 0:1] + ht_ref[:, 1:2])
        o_ref[0] = xw_ref[0] * dinv

    return pl.pallas_call(
        body,
        grid=(2, NBR),
        in_specs=[
            pl.BlockSpec((1, BR, 32), lambda p, i: (p, i, 0)),
            pl.BlockSpec((BR, 2), lambda p, i: (i, 0)),
        ],
        out_specs=pl.BlockSpec((1, BR, 32), lambda p, i: (p, i, 0)),
        out_shape=jax.ShapeDtypeStruct((2, NP, 32), F32),
    )(xw1, hist_t)


def _conv_relu_stats(S, y, hist_t, b_r, parts):
    def body(s_ref, y_ref, ht_ref, b_ref, h_ref, st_ref):
        i = pl.program_id(1)
        dinv = lax.rsqrt(1.0 + ht_ref[:, 0:1] + ht_ref[:, 1:2])
        h = jnp.maximum(dinv * (s_ref[0] + y_ref[0]) + b_ref[0], 0.0)
        h_ref[0] = h
        rows = i * BR + lax.broadcasted_iota(jnp.int32, (BR, 1), 0)
        hm = jnp.where(rows < N, h, 0.0)
        s1 = jnp.sum(hm, axis=0, keepdims=True)
        s2 = jnp.sum(hm * hm, axis=0, keepdims=True)

        @pl.when(i == 0)
        def _():
            st_ref[0] = jnp.zeros((2, 32), F32)

        st_ref[0] += jnp.concatenate([s1, s2], axis=0)

    return pl.pallas_call(
        body,
        grid=(parts, NBR),
        in_specs=[
            pl.BlockSpec((1, BR, 32), lambda p, i: (p, i, 0)),
            pl.BlockSpec((1, BR, 32), lambda p, i: (p, i, 0)),
            pl.BlockSpec((BR, 2), lambda p, i: (i, 0)),
            pl.BlockSpec((1, 1, 32), lambda p, i: (p, 0, 0)),
        ],
        out_specs=[
            pl.BlockSpec((1, BR, 32), lambda p, i: (p, i, 0)),
            pl.BlockSpec((1, 2, 32), lambda p, i: (p, 0, 0)),
        ],
        out_shape=[
            jax.ShapeDtypeStruct((parts, NP, 32), F32),
            jax.ShapeDtypeStruct((parts, 2, 32), F32),
        ],
    )(S, y, hist_t, b_r)


def _bn_mm_y2(h_pre, st, hist_t, w2, g_r, be_r):
    def body(h_ref, st_ref, ht_ref, w_ref, g_ref, be_ref, o_ref):
        m = st_ref[0:1] * (1.0 / N)
        v = st_ref[1:2] * (1.0 / N) - m * m
        inv = lax.rsqrt(v + EPS)
        hcat = jnp.concatenate([h_ref[0], h_ref[1]], axis=1)
        h = jnp.maximum(g_ref[...] * (hcat - m) * inv + be_ref[...], 0.0)
        xw = jnp.dot(h, w_ref[0], preferred_element_type=F32)
        dinv = lax.rsqrt(1.0 + ht_ref[:, 0:1] + ht_ref[:, 1:2])
        o_ref[0] = xw * dinv

    return pl.pallas_call(
        body,
        grid=(4, NBR),
        in_specs=[
            pl.BlockSpec((2, BR, 32), lambda p, i: (0, i, 0)),
            pl.BlockSpec((2, 64), lambda p, i: (0, 0)),
            pl.BlockSpec((BR, 2), lambda p, i: (i, 0)),
            pl.BlockSpec((1, 64, 32), lambda p, i: (p, 0, 0)),
            pl.BlockSpec((1, 64), lambda p, i: (0, 0)),
            pl.BlockSpec((1, 64), lambda p, i: (0, 0)),
        ],
        out_specs=pl.BlockSpec((1, BR, 32), lambda p, i: (p, i, 0)),
        out_shape=jax.ShapeDtypeStruct((4, NP, 32), F32),
    )(h_pre, st, hist_t, w2, g_r, be_r)


def _bn_final(h_pre, st, g_r, be_r):
    def body(h_ref, st_ref, g_ref, be_ref, o_ref):
        cols = []
        for p in range(4):
            m = st_ref[p, 0:1] * (1.0 / N)
            v = st_ref[p, 1:2] * (1.0 / N) - m * m
            inv = lax.rsqrt(v + EPS)
            cols.append(g_ref[p] * (h_ref[p] - m) * inv + be_ref[p])
        o_ref[...] = jnp.concatenate(cols, axis=1)

    return pl.pallas_call(
        body,
        grid=(-(-N // BR),),
        in_specs=[
            pl.BlockSpec((4, BR, 32), lambda i: (0, i, 0)),
            pl.BlockSpec((4, 2, 32), lambda i: (0, 0, 0)),
            pl.BlockSpec((4, 1, 32), lambda i: (0, 0, 0)),
            pl.BlockSpec((4, 1, 32), lambda i: (0, 0, 0)),
        ],
        out_specs=pl.BlockSpec((BR, 128), lambda i: (i, 0)),
        out_shape=jax.ShapeDtypeStruct((N, 128), F32),
    )(h_pre, st, g_r, be_r)


def kernel(x, edge_index, W1, b1, g1, be1, W2, b2, g2, be2):
    src = edge_index[0]
    dst = edge_index[1]
    pad = EP - E
    ar = jnp.arange(pad, dtype=dst.dtype)
    srcm = jnp.concatenate([src, ar % N]).reshape(NCHP, CH)
    dstm = jnp.concatenate([dst, N + (ar % DUMP)]).reshape(NCHP, CH)
    x_p = jnp.pad(x, ((0, NP - N), (0, 0)))

    w1p = W1.reshape(32, 2, 32).transpose(1, 0, 2)
    w2p = W2.reshape(64, 4, 32).transpose(1, 0, 2)

    hist = _deg_hist(dstm)
    xw1 = _xw1(x_p, w1p)
    hist_t = hist.reshape(2, NP).T

    y1 = _scale_y1(xw1, hist_t)
    S1 = _segment_sum(y1, srcm, dstm, 2)
    h1_pre, st1 = _conv_relu_stats(S1, y1, hist_t,
                                   b1.reshape(2, 1, 32), 2)
    st1r = st1.transpose(1, 0, 2).reshape(2, 64)
    y2 = _bn_mm_y2(h1_pre, st1r, hist_t, w2p,
                   g1.reshape(1, 64), be1.reshape(1, 64))
    S2 = _segment_sum(y2, srcm, dstm, 4)
    h2_pre, st2 = _conv_relu_stats(S2, y2, hist_t,
                                   b2.reshape(4, 1, 32), 4)
    out = _bn_final(h2_pre, st2, g2.reshape(4, 1, 32), be2.reshape(4, 1, 32))
    return out


# speedup vs baseline: 22.6867x; 2.3430x over previous
import functools

import jax
import jax.numpy as jnp
from jax import lax
from jax.experimental import pallas as pl
from jax.experimental.pallas import tpu as pltpu
from jax.experimental.pallas import tpu_sc as plsc

N = 50000
E = 800000
NP = 51200
BR = 2560
NBR = NP // BR
RPT = NP // 16
CH = 128
RING = 4
IBLK = 20
NCHP = 6400
CPS = NCHP // 16
NIB = CPS // IBLK
DEG_CPS = NCHP // 32
DEG_NIB = DEG_CPS // IBLK
EP = NCHP * CH
DUMP = NP - N
F32 = jnp.float32
EPS = 1e-5


def _sc_mesh():
    return plsc.VectorSubcoreMesh(core_axis_name="c", subcore_axis_name="s",
                                  num_cores=2, num_subcores=16)


_SC_PARAMS = pltpu.CompilerParams(use_tc_tiling_on_sc=False)


def _deg_hist(dstm, zeros1):
    @functools.partial(
        pl.kernel,
        out_type=jax.ShapeDtypeStruct((2, 1, NP), F32),
        mesh=_sc_mesh(),
        compiler_params=_SC_PARAMS,
        scratch_types=[
            pltpu.VMEM((2, IBLK, CH), jnp.int32),
            pltpu.VMEM((CH,), F32),
            pltpu.VMEM_SHARED((NP,), F32),
            pltpu.SemaphoreType.DMA((2,)),
            pltpu.SemaphoreType.DMA((RING,)),
        ],
    )
    def k(dstm_hbm, z_hbm, hist_hbm, dbuf, ones, acc, isem, ssem):
        c = lax.axis_index("c")
        s = lax.axis_index("s")

        @pl.loop(0, CH, step=16)
        def _(i):
            ones[pl.ds(i, 16)] = jnp.ones((16,), F32)

        pltpu.sync_copy(z_hbm, acc.at[pl.ds(s * RPT, RPT)])
        plsc.subcore_barrier()

        off0 = (c * 16 + s) * DEG_CPS

        def iload(k_ib, kb):
            return pltpu.make_async_copy(
                dstm_hbm.at[pl.ds(off0 + k_ib * IBLK, IBLK)],
                dbuf.at[kb], isem.at[kb])

        def scat(kb, jj, b):
            return pltpu.make_async_copy(ones, acc.at[dbuf.at[kb, jj]],
                                         ssem.at[b])

        iload(0, 0).start()

        def blk(k_ib, kb):
            iload(k_ib, kb).wait()
            for q in range(IBLK // RING):
                for b in range(RING):
                    jj = q * RING + b
                    if q == 0:
                        @pl.when(k_ib > 0)
                        def _():
                            scat(1 - kb, IBLK - RING + b, b).wait()
                    else:
                        scat(kb, jj - RING, b).wait()
                    pltpu.async_copy(ones, acc.at[dbuf.at[kb, jj]],
                                     ssem.at[b], add=True)
                if q == 0:
                    iload(k_ib + 1, 1 - kb).start()

        @pl.loop(0, DEG_NIB, step=2)
        def _(k_ib):
            blk(k_ib, 0)
            blk(k_ib + 1, 1)

        iload(DEG_NIB, 0).wait()
        for b in range(RING):
            scat(1, IBLK - RING + b, b).wait()
        plsc.subcore_barrier()
        pltpu.sync_copy(acc.at[pl.ds(s * RPT, RPT)],
                        hist_hbm.at[c, 0, pl.ds(s * RPT, RPT)])

    return k(dstm, zeros1)


def _segment_sum(y, srcm, dstm, zeros2, parts):
    passes = parts // 2

    @functools.partial(
        pl.kernel,
        out_type=jax.ShapeDtypeStruct((parts, NP, 32), F32),
        mesh=_sc_mesh(),
        compiler_params=_SC_PARAMS,
        scratch_types=[
            pltpu.VMEM((2, IBLK, CH), jnp.int32),
            pltpu.VMEM((2, IBLK, CH), jnp.int32),
            pltpu.VMEM((RING, CH, 32), F32),
            pltpu.VMEM_SHARED((NP, 32), F32),
            pltpu.SemaphoreType.DMA((2,)),
            pltpu.SemaphoreType.DMA((2,)),
            pltpu.SemaphoreType.DMA((RING,)),
            pltpu.SemaphoreType.DMA((RING,)),
        ],
    )
    def k(y_hbm, srcm_hbm, dstm_hbm, z_hbm, out_hbm,
          sbuf, dbuf, rbuf, acc, isems, isemd, gsem, ssem):
        c = lax.axis_index("c")
        s = lax.axis_index("s")
        off0 = s * CPS

        def iload(k_ib, kb):
            return (
                pltpu.make_async_copy(
                    srcm_hbm.at[pl.ds(off0 + k_ib * IBLK, IBLK)],
                    sbuf.at[kb], isems.at[kb]),
                pltpu.make_async_copy(
                    dstm_hbm.at[pl.ds(off0 + k_ib * IBLK, IBLK)],
                    dbuf.at[kb], isemd.at[kb]),
            )

        for p_i in range(passes):
            part = c * passes + p_i

            pltpu.sync_copy(z_hbm, acc.at[pl.ds(s * RPT, RPT)])
            plsc.subcore_barrier()

            def gath(kb, jj, b):
                return pltpu.make_async_copy(
                    y_hbm.at[part].at[sbuf.at[kb, jj]], rbuf.at[b],
                    gsem.at[b])

            def scat(kb, jj, b):
                return pltpu.make_async_copy(rbuf.at[b],
                                             acc.at[dbuf.at[kb, jj]],
                                             ssem.at[b])

            for d in iload(0, 0):
                d.start()

            def blk(k_ib, kb):
                for d in iload(k_ib, kb):
                    d.wait()
                for q in range(IBLK // RING):
                    for b in range(RING):
                        jj = q * RING + b
                        if q == 0:
                            @pl.when(k_ib > 0)
                            def _():
                                scat(1 - kb, IBLK - RING + b, b).wait()
                        else:
                            scat(kb, jj - RING, b).wait()
                        gath(kb, jj, b).start()
                    if q == 0:
                        for d in iload(k_ib + 1, 1 - kb):
                            d.start()
                    for b in range(RING):
                        jj = q * RING + b
                        gath(kb, jj, b).wait()
                        pltpu.async_copy(rbuf.at[b], acc.at[dbuf.at[kb, jj]],
                                         ssem.at[b], add=True)

            @pl.loop(0, NIB, step=2)
            def _(k_ib):
                blk(k_ib, 0)
                blk(k_ib + 1, 1)

            for d in iload(NIB, 0):
                d.wait()
            for b in range(RING):
                scat(1, IBLK - RING + b, b).wait()
            plsc.subcore_barrier()
            pltpu.sync_copy(acc.at[pl.ds(s * RPT, RPT)],
                            out_hbm.at[part, pl.ds(s * RPT, RPT)])

    return k(y, srcm, dstm, zeros2)


def _xw1(x_p, w1p):
    def body(x_ref, w_ref, o_ref):
        for p in range(2):
            o_ref[p] = jnp.dot(x_ref[...], w_ref[p],
                               preferred_element_type=F32)

    return pl.pallas_call(
        body,
        grid=(NBR,),
        in_specs=[
            pl.BlockSpec((BR, 32), lambda i: (i, 0)),
            pl.BlockSpec((2, 32, 32), lambda i: (0, 0, 0)),
        ],
        out_specs=pl.BlockSpec((2, BR, 32), lambda i: (0, i, 0)),
        out_shape=jax.ShapeDtypeStruct((2, NP, 32), F32),
    )(x_p, w1p)


def _dinv_row(hist):
    def body(h_ref, o_ref):
        o_ref[...] = lax.rsqrt(1.0 + h_ref[0] + h_ref[1])

    return pl.pallas_call(
        body,
        grid=(1,),
        in_specs=[pl.BlockSpec((2, 1, NP), lambda i: (0, 0, 0))],
        out_specs=pl.BlockSpec((1, NP), lambda i: (0, 0)),
        out_shape=jax.ShapeDtypeStruct((1, NP), F32),
    )(hist)


def _scale_y1(xw1, dinv_c):
    def body(xw_ref, d_ref, o_ref):
        for p in range(2):
            o_ref[p] = xw_ref[p] * d_ref[...]

    return pl.pallas_call(
        body,
        grid=(NBR,),
        in_specs=[
            pl.BlockSpec((2, BR, 32), lambda i: (0, i, 0)),
            pl.BlockSpec((BR, 1), lambda i: (i, 0)),
        ],
        out_specs=pl.BlockSpec((2, BR, 32), lambda i: (0, i, 0)),
        out_shape=jax.ShapeDtypeStruct((2, NP, 32), F32),
    )(xw1, dinv_c)


def _conv_relu_stats(S, y, dinv_c, b_r, parts):
    def body(s_ref, y_ref, d_ref, b_ref, h_ref, st_ref):
        i = pl.program_id(0)

        @pl.when(i == 0)
        def _():
            st_ref[...] = jnp.zeros((parts, 2, 32), F32)

        rows = i * BR + lax.broadcasted_iota(jnp.int32, (BR, 1), 0)
        msk = rows < N
        d = d_ref[...]
        for p in range(parts):
            h = jnp.maximum(d * (s_ref[p] + y_ref[p]) + b_ref[p], 0.0)
            h_ref[p] = h
            hm = jnp.where(msk, h, 0.0)
            s1 = jnp.sum(hm, axis=0, keepdims=True)
            s2 = jnp.sum(hm * hm, axis=0, keepdims=True)
            st_ref[p] += jnp.concatenate([s1, s2], axis=0)

    return pl.pallas_call(
        body,
        grid=(NBR,),
        in_specs=[
            pl.BlockSpec((parts, BR, 32), lambda i: (0, i, 0)),
            pl.BlockSpec((parts, BR, 32), lambda i: (0, i, 0)),
            pl.BlockSpec((BR, 1), lambda i: (i, 0)),
            pl.BlockSpec((parts, 1, 32), lambda i: (0, 0, 0)),
        ],
        out_specs=[
            pl.BlockSpec((parts, BR, 32), lambda i: (0, i, 0)),
            pl.BlockSpec((parts, 2, 32), lambda i: (0, 0, 0)),
        ],
        out_shape=[
            jax.ShapeDtypeStruct((parts, NP, 32), F32),
            jax.ShapeDtypeStruct((parts, 2, 32), F32),
        ],
    )(S, y, dinv_c, b_r)


def _bn_mm_y2(h_pre, st, dinv_c, w2p, g_r, be_r):
    def body(h_ref, st_ref, d_ref, w_ref, g_ref, be_ref, o_ref):
        cols = []
        for p in range(2):
            m = st_ref[p, 0:1] * (1.0 / N)
            v = st_ref[p, 1:2] * (1.0 / N) - m * m
            inv = lax.rsqrt(v + EPS)
            cols.append(jnp.maximum(
                g_ref[p] * (h_ref[p] - m) * inv + be_ref[p], 0.0))
        h = jnp.concatenate(cols, axis=1)
        d = d_ref[...]
        for p in range(4):
            o_ref[p] = jnp.dot(h, w_ref[p], preferred_element_type=F32) * d

    return pl.pallas_call(
        body,
        grid=(NBR,),
        in_specs=[
            pl.BlockSpec((2, BR, 32), lambda i: (0, i, 0)),
            pl.BlockSpec((2, 2, 32), lambda i: (0, 0, 0)),
            pl.BlockSpec((BR, 1), lambda i: (i, 0)),
            pl.BlockSpec((4, 64, 32), lambda i: (0, 0, 0)),
            pl.BlockSpec((2, 1, 32), lambda i: (0, 0, 0)),
            pl.BlockSpec((2, 1, 32), lambda i: (0, 0, 0)),
        ],
        out_specs=pl.BlockSpec((4, BR, 32), lambda i: (0, i, 0)),
        out_shape=jax.ShapeDtypeStruct((4, NP, 32), F32),
    )(h_pre, st, dinv_c, w2p, g_r, be_r)


BRF = 2000


def _bn_final(h_pre, st, g_r, be_r):
    def body(h_ref, st_ref, g_ref, be_ref, o_ref):
        cols = []
        for p in range(4):
            m = st_ref[p, 0:1] * (1.0 / N)
            v = st_ref[p, 1:2] * (1.0 / N) - m * m
            inv = lax.rsqrt(v + EPS)
            cols.append(g_ref[p] * (h_ref[p] - m) * inv + be_ref[p])
        o_ref[...] = jnp.concatenate(cols, axis=1)

    return pl.pallas_call(
        body,
        grid=(N // BRF,),
        in_specs=[
            pl.BlockSpec((4, BRF, 32), lambda i: (0, i, 0)),
            pl.BlockSpec((4, 2, 32), lambda i: (0, 0, 0)),
            pl.BlockSpec((4, 1, 32), lambda i: (0, 0, 0)),
            pl.BlockSpec((4, 1, 32), lambda i: (0, 0, 0)),
        ],
        out_specs=pl.BlockSpec((BRF, 128), lambda i: (i, 0)),
        out_shape=jax.ShapeDtypeStruct((N, 128), F32),
    )(h_pre, st, g_r, be_r)


def kernel(x, edge_index, W1, b1, g1, be1, W2, b2, g2, be2):
    src = edge_index[0]
    dst = edge_index[1]
    pad = EP - E
    ar = jnp.arange(pad, dtype=dst.dtype)
    xpad = jnp.zeros((IBLK * CH,), dst.dtype)
    srcm = jnp.concatenate([src, ar % N, xpad]).reshape(NCHP + IBLK, CH)
    dstm = jnp.concatenate([dst, N + (ar % DUMP),
                            xpad + N]).reshape(NCHP + IBLK, CH)
    x_p = jnp.pad(x, ((0, NP - N), (0, 0)))
    w1p = W1.reshape(32, 2, 32).transpose(1, 0, 2)
    w2p = W2.reshape(64, 4, 32).transpose(1, 0, 2)
    zeros1 = jnp.zeros((RPT,), F32)
    zeros2 = jnp.zeros((RPT, 32), F32)

    hist = _deg_hist(dstm, zeros1)
    xw1 = _xw1(x_p, w1p)
    dinv_c = _dinv_row(hist).reshape(NP, 1)

    y1 = _scale_y1(xw1, dinv_c)
    S1 = _segment_sum(y1, srcm, dstm, zeros2, 2)
    h1_pre, st1 = _conv_relu_stats(S1, y1, dinv_c,
                                   b1.reshape(2, 1, 32), 2)
    y2 = _bn_mm_y2(h1_pre, st1, dinv_c, w2p,
                   g1.reshape(2, 1, 32), be1.reshape(2, 1, 32))
    S2 = _segment_sum(y2, srcm, dstm, zeros2, 4)
    h2_pre, st2 = _conv_relu_stats(S2, y2, dinv_c,
                                   b2.reshape(4, 1, 32), 4)
    out = _bn_final(h2_pre, st2, g2.reshape(4, 1, 32), be2.reshape(4, 1, 32))
    return out
